# baseline (speedup 1.0000x reference)
import functools

import jax
import jax.numpy as jnp
from jax import lax
from jax.experimental import pallas as pl
from jax.experimental.pallas import tpu as pltpu
from jax.experimental.pallas import tpu_sc as plsc

N = 10000
NPAD = 10240
F = 128
E = 320000
NC, NS = 2, 16
NW = NC * NS
RB = NPAD // NW
CHUNK = 128
EPW = 10240
EPAD = NW * EPW
SLAB = EPW + NW * 8 + 128
BUF = NW * SLAB + 128

_mesh = plsc.VectorSubcoreMesh(
    core_axis_name="c", subcore_axis_name="s", num_cores=NC, num_subcores=NS
)

_i32 = jnp.int32
_f32 = jnp.float32


@functools.partial(
    pl.kernel,
    out_type=[
        jax.ShapeDtypeStruct((BUF,), _i32),
        jax.ShapeDtypeStruct((BUF,), _i32),
        jax.ShapeDtypeStruct((NW, 128), _i32),
    ],
    mesh=_mesh,
    scratch_types=[
        pltpu.VMEM((EPW,), _i32),
        pltpu.VMEM((EPW,), _i32),
        pltpu.VMEM((EPW,), _i32),
        pltpu.VMEM((4, CHUNK), _i32),
        pltpu.VMEM((128,), _i32),
        pltpu.VMEM((NW + 16,), _i32),
        pltpu.SemaphoreType.DMA((4,)),
        pltpu.SemaphoreType.DMA((4,)),
    ],
)
def _sc_binsort(src_hbm, dst_hbm, bsrc, bdst, cnt_out,
                srcall, dstall, binv, posb, hist, cur, semP, semQ):
    c = lax.axis_index("c")
    s = lax.axis_index("s")
    w = c * NS + s
    pltpu.sync_copy(src_hbm.at[w], srcall)
    pltpu.sync_copy(dst_hbm.at[w], dstall)

    @pl.loop(0, EPW // 16)
    def _(q):
        v = dstall[pl.ds(q * 16, 16)]
        binv[pl.ds(q * 16, 16)] = ((v >> 6) * 205) >> 10

    lane = lax.iota(_i32, 16)
    onehot0 = jnp.where(lane == 0, jnp.int32(1), jnp.int32(0))
    zero16 = jnp.zeros((16,), _i32)
    for i in range(8):
        hist[pl.ds(i * 16, 16)] = zero16

    @pl.loop(0, EPW // 16)
    def _(q):
        bv = binv[pl.ds(q * 16, 16)]
        for j in range(16):
            b = bv[j]
            hist[pl.ds(b, 16)] = hist[pl.ds(b, 16)] + onehot0

    pltpu.sync_copy(hist, cnt_out.at[w])

    h_lo = hist[pl.ds(0, 16)]
    h_hi = hist[pl.ds(16, 16)]
    cur_lo = zero16
    cur_hi = zero16
    off = w * SLAB
    for b in range(NW):
        t = h_lo[b] if b < 16 else h_hi[b - 16]
        if b < 16:
            cur_lo = jnp.where(lane == b, off, cur_lo)
        else:
            cur_hi = jnp.where(lane == (b - 16), off, cur_hi)
        off = off + (((t + 7) >> 3) << 3)
    cur[pl.ds(0, 16)] = cur_lo
    cur[pl.ds(16, 16)] = cur_hi
    cur[pl.ds(32, 16)] = zero16

    def do_chunk(k, slot):
        for g in range(CHUNK // 16):
            bv = binv[pl.ds(k * CHUNK + g * 16, 16)]
            posv = zero16
            for j in range(16):
                b = bv[j]
                wnd = cur[pl.ds(b, 16)]
                p = wnd[0]
                cur[pl.ds(b, 16)] = wnd + onehot0
                posv = jnp.where(lane == j, p, posv)
            posb[slot, pl.ds(g * 16, 16)] = posv
        pltpu.async_copy(srcall.at[pl.ds(k * CHUNK, CHUNK)],
                         bsrc.at[posb.at[slot]], semP.at[slot])
        pltpu.async_copy(dstall.at[pl.ds(k * CHUNK, CHUNK)],
                         bdst.at[posb.at[slot]], semQ.at[slot])

    def drain(slot):
        pltpu.make_async_copy(srcall.at[pl.ds(0, CHUNK)],
                              bsrc.at[posb.at[slot]], semP.at[slot]).wait()
        pltpu.make_async_copy(dstall.at[pl.ds(0, CHUNK)],
                              bdst.at[posb.at[slot]], semQ.at[slot]).wait()

    for slot in range(4):
        do_chunk(jnp.int32(slot), slot)

    @pl.loop(1, EPW // CHUNK // 4)
    def _(o):
        for slot in range(4):
            drain(slot)
            do_chunk(o * 4 + slot, slot)

    for slot in range(4):
        drain(slot)


def _seg_loop(cntv, w, per_segment_body):
    @pl.loop(0, NW)
    def _(w2):
        @pl.loop(0, w, init_carry=jnp.int32(0))
        def segoff(b2, acc):
            tt = cntv[pl.ds(w2 * 128 + b2, 16)][0]
            return acc + (((tt + 7) >> 3) << 3)

        t = cntv[pl.ds(w2 * 128 + w, 16)][0]
        base = w2 * SLAB if segoff is None else (w2 * SLAB + segoff)
        per_segment_body(pl.multiple_of(base, 8), t)


@functools.partial(
    pl.kernel,
    out_type=jax.ShapeDtypeStruct((NPAD, 16), _f32),
    mesh=_mesh,
    scratch_types=[
        pltpu.VMEM((CHUNK + 16,), _i32),
        pltpu.VMEM((RB + 8, 16), _f32),
        pltpu.VMEM((NW * 128 + 16,), _i32),
    ],
)
def _sc_degree(bdst_hbm, cnt_hbm, deg_out, didx, degp, cntv):
    c = lax.axis_index("c")
    s = lax.axis_index("s")
    w = c * NS + s
    pltpu.sync_copy(cnt_hbm, cntv.at[pl.ds(0, NW * 128)])

    @pl.loop(0, RB + 8)
    def _(r):
        degp[r, pl.ds(0, 16)] = jnp.zeros((16,), _f32)

    nodebase = w * RB
    lane = lax.iota(_i32, 16)

    def do_segment(base, t):
        @pl.loop(0, (t + CHUNK - 1) >> 7)
        def _(k):
            pltpu.sync_copy(bdst_hbm.at[pl.ds(base + k * CHUNK, CHUNK)],
                            didx.at[pl.ds(0, CHUNK)])
            rem = jnp.minimum(t - k * CHUNK, CHUNK)

            @pl.loop(0, CHUNK // 16)
            def _(q):
                dv = didx[pl.ds(q * 16, 16)] - nodebase
                dv = jnp.minimum(jnp.maximum(dv, 0), jnp.int32(RB))
                dv = jnp.where(lane + q * 16 < rem, dv, jnp.int32(RB))
                for j in range(16):
                    l = dv[j]
                    degp[l, pl.ds(0, 16)] = degp[l, pl.ds(0, 16)] + 1.0

    _seg_loop(cntv, w, do_segment)
    pltpu.sync_copy(degp.at[pl.ds(0, RB)], deg_out.at[pl.ds(nodebase, RB)])


NCHMAX = 2624


@functools.partial(
    pl.kernel,
    out_type=jax.ShapeDtypeStruct((8, NPAD * 16), _f32),
    mesh=_mesh,
    scratch_types=[
        pltpu.VMEM((4, CHUNK), _i32),
        pltpu.VMEM((4, CHUNK), _i32),
        pltpu.VMEM((2, CHUNK, F), _f32),
    ] + [pltpu.VMEM(((RB + 8) * 16,), _f32)] * 8 + [
        pltpu.VMEM((NW * 128 + 16,), _i32),
        pltpu.VMEM((NCHMAX,), _i32),
        pltpu.VMEM((NCHMAX,), _i32),
        pltpu.SemaphoreType.DMA((4,)),
        pltpu.SemaphoreType.DMA((4,)),
        pltpu.SemaphoreType.DMA((2,)),
    ],
)
def _sc_scatter(g_hbm, bsrc_hbm, bdst_hbm, cnt_hbm, zcol_hbm,
                s_out, sidxr, didxr, rows2,
                acc0, acc1, acc2, acc3, acc4, acc5, acc6, acc7,
                cntv, cb, cr, semI, semJ, semG):
    c = lax.axis_index("c")
    s = lax.axis_index("s")
    w = c * NS + s
    accs = (acc0, acc1, acc2, acc3, acc4, acc5, acc6, acc7)
    pltpu.sync_copy(cnt_hbm, cntv.at[pl.ds(0, NW * 128)])
    for f in range(8):
        pltpu.sync_copy(zcol_hbm, accs[f])

    nodebase = w * RB
    lane = lax.iota(_i32, 16)

    @pl.loop(0, NW, init_carry=jnp.int32(0))
    def build(w2, cursor):
        @pl.loop(0, w, init_carry=jnp.int32(0))
        def segoff(b2, o):
            tt = cntv[pl.ds(w2 * 128 + b2, 16)][0]
            return o + (((tt + 7) >> 3) << 3)

        t = cntv[pl.ds(w2 * 128 + w, 16)][0]
        base = w2 * SLAB + segoff
        nch = (t + CHUNK - 1) >> 7

        @pl.loop(0, (nch + 15) >> 4)
        def _(gi):
            kv = gi * 16 + lane
            cb[pl.ds(cursor + gi * 16, 16)] = base + kv * CHUNK
            cr[pl.ds(cursor + gi * 16, 16)] = jnp.minimum(
                jnp.maximum(t - kv * CHUNK, 0), jnp.int32(CHUNK))

        return cursor + nch

    ncht = build

    def issue_idx(j, k):
        b = pl.multiple_of(cb[pl.ds(j, 16)][0], 8)
        pltpu.async_copy(bsrc_hbm.at[pl.ds(b, CHUNK)], sidxr.at[k], semI.at[k])
        pltpu.async_copy(bdst_hbm.at[pl.ds(b, CHUNK)], didxr.at[k], semJ.at[k])

    def wait_idx_issue_gather(k, r):
        pltpu.make_async_copy(bsrc_hbm.at[pl.ds(0, CHUNK)], sidxr.at[k],
                              semI.at[k]).wait()
        pltpu.make_async_copy(bdst_hbm.at[pl.ds(0, CHUNK)], didxr.at[k],
                              semJ.at[k]).wait()
        for q in range(CHUNK // 16):
            v = sidxr[k, pl.ds(q * 16, 16)]
            sidxr[k, pl.ds(q * 16, 16)] = jnp.minimum(
                jnp.maximum(v, 0), jnp.int32(NPAD - 1))
        pltpu.async_copy(g_hbm.at[sidxr.at[k]], rows2.at[r], semG.at[r])

    def process(j, k, r):
        pltpu.make_async_copy(g_hbm.at[sidxr.at[k]], rows2.at[r],
                              semG.at[r]).wait()
        rem = cr[pl.ds(j, 16)][0]

        @pl.loop(0, CHUNK // 16)
        def _(q):
            dv = didxr[k, pl.ds(q * 16, 16)] - nodebase
            dv = jnp.minimum(jnp.maximum(dv, 0), jnp.int32(RB))
            dv = jnp.where(lane + q * 16 < rem, dv, jnp.int32(RB))
            for jj in range(16):
                l = dv[jj]
                e = q * 16 + jj
                li = l * 16
                for f in range(F // 16):
                    accs[f][pl.ds(li, 16)] = (
                        accs[f][pl.ds(li, 16)]
                        + rows2[r, e, pl.ds(f * 16, 16)])

    for k in range(4):
        @pl.when(k < ncht)
        def _(k=k):
            issue_idx(k, k)
    for r in range(2):
        @pl.when(r < ncht)
        def _(r=r):
            wait_idx_issue_gather(r, r)

    @pl.loop(0, (ncht + 3) >> 2)
    def _(o):
        for ph in range(4):
            j = o * 4 + ph

            @pl.when(j < ncht)
            def _(j=j, ph=ph):
                process(j, ph, ph % 2)

                @pl.when(j + 4 < ncht)
                def _():
                    issue_idx(j + 4, ph)

                @pl.when(j + 2 < ncht)
                def _():
                    wait_idx_issue_gather((ph + 2) % 4, ph % 2)

    for f in range(8):
        pltpu.sync_copy(accs[f].at[pl.ds(0, RB * 16)],
                        s_out.at[f, pl.ds(nodebase * 16, RB * 16)])


@functools.partial(
    pl.kernel,
    out_type=jax.ShapeDtypeStruct((1024, F), _f32),
    mesh=_mesh,
    scratch_types=[
        pltpu.VMEM((32, F), _f32),
        pltpu.VMEM((32,), _i32),
        pltpu.SemaphoreType.DMA,
    ],
)
def _sc_gather_batch(feat_hbm, bidx_hbm, out_hbm, rows, bv, sem):
    c = lax.axis_index("c")
    s = lax.axis_index("s")
    pltpu.sync_copy(bidx_hbm.at[c, s], bv)
    pltpu.async_copy(feat_hbm.at[bv], rows, sem).wait()
    pltpu.sync_copy(rows, out_hbm.at[pl.ds(c * 512 + s * 32, 32)])


def _tc_first(deg, x, W1):
    def body(deg_ref, x_ref, w_ref, dis_ref, g_ref):
        i = pl.program_id(0)
        row = i * 128 + lax.broadcasted_iota(_i32, (128, 1), 0)
        dis = jnp.where(row < N, lax.rsqrt(deg_ref[:, 0:1] + 1.0), 0.0)
        dis_ref[...] = dis
        g_ref[...] = jnp.dot(x_ref[...], w_ref[...],
                             preferred_element_type=_f32) * dis

    return pl.pallas_call(
        body,
        grid=(NPAD // 128,),
        in_specs=[
            pl.BlockSpec((128, 16), lambda i: (i, 0)),
            pl.BlockSpec((128, F), lambda i: (i, 0)),
            pl.BlockSpec((F, F), lambda i: (0, 0)),
        ],
        out_specs=[
            pl.BlockSpec((128, 1), lambda i: (i, 0)),
            pl.BlockSpec((128, F), lambda i: (i, 0)),
        ],
        out_shape=[
            jax.ShapeDtypeStruct((NPAD, 1), _f32),
            jax.ShapeDtypeStruct((NPAD, F), _f32),
        ],
    )(deg, x, W1)


def _tc_mid(sacc, g, dis, b, W):
    def body(s_ref, g_ref, dis_ref, b_ref, w_ref, out_ref):
        ssum = jnp.concatenate([s_ref[f] for f in range(8)], axis=1)
        h = jax.nn.relu(dis_ref[...] * (ssum + g_ref[...]) + b_ref[...])
        out_ref[...] = jnp.dot(h, w_ref[...],
                               preferred_element_type=_f32) * dis_ref[...]

    return pl.pallas_call(
        body,
        grid=(NPAD // 128,),
        in_specs=[
            pl.BlockSpec((8, 128, 16), lambda i: (0, i, 0)),
            pl.BlockSpec((128, F), lambda i: (i, 0)),
            pl.BlockSpec((128, 1), lambda i: (i, 0)),
            pl.BlockSpec((1, F), lambda i: (0, 0)),
            pl.BlockSpec((F, F), lambda i: (0, 0)),
        ],
        out_specs=pl.BlockSpec((128, F), lambda i: (i, 0)),
        out_shape=jax.ShapeDtypeStruct((NPAD, F), _f32),
    )(sacc, g, dis, b, W)


def _tc_last(sacc, g, dis, b):
    def body(s_ref, g_ref, dis_ref, b_ref, out_ref):
        ssum = jnp.concatenate([s_ref[f] for f in range(8)], axis=1)
        out_ref[...] = jax.nn.relu(
            dis_ref[...] * (ssum + g_ref[...]) + b_ref[...])

    return pl.pallas_call(
        body,
        grid=(NPAD // 128,),
        in_specs=[
            pl.BlockSpec((8, 128, 16), lambda i: (0, i, 0)),
            pl.BlockSpec((128, F), lambda i: (i, 0)),
            pl.BlockSpec((128, 1), lambda i: (i, 0)),
            pl.BlockSpec((1, F), lambda i: (0, 0)),
        ],
        out_specs=pl.BlockSpec((128, F), lambda i: (i, 0)),
        out_shape=jax.ShapeDtypeStruct((NPAD, F), _f32),
    )(sacc, g, dis, b)


def _tc_head(feats, Wlin, blin):
    def body(f_ref, w_ref, b_ref, out_ref, logp_ref):
        t = jax.nn.relu(jnp.dot(f_ref[...], w_ref[...],
                                preferred_element_type=_f32) + b_ref[...])
        col = lax.broadcasted_iota(_i32, (1, 128), 1)
        valid = col < 10
        mx = jnp.max(jnp.where(valid, t, -1e30), axis=1, keepdims=True)
        ex = jnp.where(valid, jnp.exp(t - mx), 0.0)
        lse = jnp.log(jnp.sum(ex, axis=1, keepdims=True))
        out_ref[...] = t
        logp_ref[...] = t - mx - lse

    return pl.pallas_call(
        body,
        grid=(1024 // 128,),
        in_specs=[
            pl.BlockSpec((128, F), lambda i: (i, 0)),
            pl.BlockSpec((F, 128), lambda i: (0, 0)),
            pl.BlockSpec((1, 128), lambda i: (0, 0)),
        ],
        out_specs=[
            pl.BlockSpec((128, 128), lambda i: (i, 0)),
            pl.BlockSpec((128, 128), lambda i: (i, 0)),
        ],
        out_shape=[
            jax.ShapeDtypeStruct((1024, 128), _f32),
            jax.ShapeDtypeStruct((1024, 128), _f32),
        ],
    )(feats, Wlin, blin)


def kernel(x, edge_index, batch_index, W1, b1, W2, b2, Wlin, blin):
    src = edge_index[0].astype(_i32)
    dst = edge_index[1].astype(_i32)
    pad = jnp.full((EPAD - E,), N, dtype=_i32)
    srcf = jnp.concatenate([src, pad]).reshape(NW, EPW)
    dstf = jnp.concatenate([dst, pad]).reshape(NW, EPW)
    xp = jnp.concatenate([x, jnp.zeros((NPAD - N, F), dtype=_f32)])
    bidx = batch_index.astype(_i32).reshape(NC, NS, 32)
    zcol = jnp.zeros(((RB + 8) * 16,), dtype=_f32)
    Wlp = jnp.concatenate(
        [Wlin, jnp.zeros((F, 128 - Wlin.shape[1]), dtype=_f32)], axis=1)
    blp = jnp.concatenate(
        [blin, jnp.zeros((128 - blin.shape[0],), dtype=_f32)]).reshape(1, 128)
    b1r = b1.reshape(1, F)
    b2r = b2.reshape(1, F)

    bsrc, bdst, cnt = _sc_binsort(srcf, dstf)
    cntf = cnt.reshape(NW * 128)
    deg = _sc_degree(bdst, cntf)
    dis, g1 = _tc_first(deg, xp, W1)
    s1 = _sc_scatter(g1, bsrc, bdst, cntf, zcol).reshape(8, NPAD, 16)
    g2 = _tc_mid(s1, g1, dis, b1r, W2)
    s2 = _sc_scatter(g2, bsrc, bdst, cntf, zcol).reshape(8, NPAD, 16)
    feats = _tc_last(s2, g2, dis, b2r)
    feats_sel = _sc_gather_batch(feats, bidx)
    outp, logpp = _tc_head(feats_sel, Wlp, blp)
    return (logpp[:, :10], outp[:, :10], feats_sel)


# speedup vs baseline: 1.1126x; 1.1126x over previous
import functools

import jax
import jax.numpy as jnp
from jax import lax
from jax.experimental import pallas as pl
from jax.experimental.pallas import tpu as pltpu
from jax.experimental.pallas import tpu_sc as plsc

N = 10000
NPAD = 10240
F = 128
E = 320000
NC, NS = 2, 16
NW = NC * NS
RB = NPAD // NW
CHUNK = 128
EPW = 10240
EPAD = NW * EPW
SLAB = EPW + NW * 8 + 128
BUF = NW * SLAB + 128

_mesh = plsc.VectorSubcoreMesh(
    core_axis_name="c", subcore_axis_name="s", num_cores=NC, num_subcores=NS
)

_i32 = jnp.int32
_f32 = jnp.float32


@functools.partial(
    pl.kernel,
    out_type=[
        jax.ShapeDtypeStruct((BUF,), _i32),
        jax.ShapeDtypeStruct((BUF,), _i32),
        jax.ShapeDtypeStruct((NW, 128), _i32),
    ],
    mesh=_mesh,
    scratch_types=[
        pltpu.VMEM((EPW,), _i32),
        pltpu.VMEM((EPW,), _i32),
        pltpu.VMEM((EPW,), _i32),
        pltpu.VMEM((EPW,), _i32),
        pltpu.VMEM((4, CHUNK), _i32),
        pltpu.VMEM((128,), _i32),
        pltpu.SemaphoreType.DMA((4,)),
        pltpu.SemaphoreType.DMA((4,)),
    ],
)
def _sc_binsort(src_hbm, dst_hbm, bsrc, bdst, cnt_out,
                srcall, dstall, binv, rankv, posb, hist, semP, semQ):
    c = lax.axis_index("c")
    s = lax.axis_index("s")
    w = c * NS + s
    pltpu.sync_copy(src_hbm.at[w], srcall)
    pltpu.sync_copy(dst_hbm.at[w], dstall)

    @pl.loop(0, EPW // 16)
    def _(q):
        v = dstall[pl.ds(q * 16, 16)]
        binv[pl.ds(q * 16, 16)] = ((v >> 6) * 205) >> 10

    lane = lax.iota(_i32, 16)
    onehot0 = jnp.where(lane == 0, jnp.int32(1), jnp.int32(0))
    zero16 = jnp.zeros((16,), _i32)
    for i in range(8):
        hist[pl.ds(i * 16, 16)] = zero16

    @pl.loop(0, EPW // 16)
    def _(q):
        bv = binv[pl.ds(q * 16, 16)]
        rkv = zero16
        for j in range(16):
            b = bv[j]
            wnd = hist[pl.ds(b, 16)]
            rkv = jnp.where(lane == j, wnd[0], rkv)
            hist[pl.ds(b, 16)] = wnd + onehot0
        rankv[pl.ds(q * 16, 16)] = rkv

    pltpu.sync_copy(hist, cnt_out.at[w])

    h_lo = hist[pl.ds(0, 16)]
    h_hi = hist[pl.ds(16, 16)]
    bases = []
    off = w * SLAB
    for b in range(NW):
        t = h_lo[b] if b < 16 else h_hi[b - 16]
        bases.append(off)
        off = off + (((t + 7) >> 3) << 3)

    def do_chunk(k, slot):
        for g in range(CHUNK // 16):
            bv = binv[pl.ds(k * CHUNK + g * 16, 16)]
            posv = rankv[pl.ds(k * CHUNK + g * 16, 16)]
            for b in range(NW):
                posv = jnp.where(bv == b, posv + bases[b], posv)
            posb[slot, pl.ds(g * 16, 16)] = posv
        pltpu.async_copy(srcall.at[pl.ds(k * CHUNK, CHUNK)],
                         bsrc.at[posb.at[slot]], semP.at[slot])
        pltpu.async_copy(dstall.at[pl.ds(k * CHUNK, CHUNK)],
                         bdst.at[posb.at[slot]], semQ.at[slot])

    def drain(slot):
        pltpu.make_async_copy(srcall.at[pl.ds(0, CHUNK)],
                              bsrc.at[posb.at[slot]], semP.at[slot]).wait()
        pltpu.make_async_copy(dstall.at[pl.ds(0, CHUNK)],
                              bdst.at[posb.at[slot]], semQ.at[slot]).wait()

    for slot in range(4):
        do_chunk(jnp.int32(slot), slot)

    @pl.loop(1, EPW // CHUNK // 4)
    def _(o):
        for slot in range(4):
            drain(slot)
            do_chunk(o * 4 + slot, slot)

    for slot in range(4):
        drain(slot)


def _seg_loop(cntv, w, per_segment_body):
    @pl.loop(0, NW)
    def _(w2):
        @pl.loop(0, w, init_carry=jnp.int32(0))
        def segoff(b2, acc):
            tt = cntv[pl.ds(w2 * 128 + b2, 16)][0]
            return acc + (((tt + 7) >> 3) << 3)

        t = cntv[pl.ds(w2 * 128 + w, 16)][0]
        base = w2 * SLAB if segoff is None else (w2 * SLAB + segoff)
        per_segment_body(pl.multiple_of(base, 8), t)


@functools.partial(
    pl.kernel,
    out_type=jax.ShapeDtypeStruct((NPAD, 16), _f32),
    mesh=_mesh,
    scratch_types=[
        pltpu.VMEM((CHUNK + 16,), _i32),
        pltpu.VMEM((RB + 8, 16), _f32),
        pltpu.VMEM((NW * 128 + 16,), _i32),
    ],
)
def _sc_degree(bdst_hbm, cnt_hbm, deg_out, didx, degp, cntv):
    c = lax.axis_index("c")
    s = lax.axis_index("s")
    w = c * NS + s
    pltpu.sync_copy(cnt_hbm, cntv.at[pl.ds(0, NW * 128)])

    @pl.loop(0, RB + 8)
    def _(r):
        degp[r, pl.ds(0, 16)] = jnp.zeros((16,), _f32)

    nodebase = w * RB
    lane = lax.iota(_i32, 16)

    def do_segment(base, t):
        @pl.loop(0, (t + CHUNK - 1) >> 7)
        def _(k):
            pltpu.sync_copy(bdst_hbm.at[pl.ds(base + k * CHUNK, CHUNK)],
                            didx.at[pl.ds(0, CHUNK)])
            rem = jnp.minimum(t - k * CHUNK, CHUNK)

            @pl.loop(0, CHUNK // 16)
            def _(q):
                dv = didx[pl.ds(q * 16, 16)] - nodebase
                dv = jnp.minimum(jnp.maximum(dv, 0), jnp.int32(RB))
                dv = jnp.where(lane + q * 16 < rem, dv, jnp.int32(RB))
                for j in range(16):
                    l = dv[j]
                    degp[l, pl.ds(0, 16)] = degp[l, pl.ds(0, 16)] + 1.0

    _seg_loop(cntv, w, do_segment)
    pltpu.sync_copy(degp.at[pl.ds(0, RB)], deg_out.at[pl.ds(nodebase, RB)])


NCHMAX = 2624


@functools.partial(
    pl.kernel,
    out_type=jax.ShapeDtypeStruct((NPAD, F), _f32),
    mesh=_mesh,
    scratch_types=[
        pltpu.VMEM((4, CHUNK), _i32),
        pltpu.VMEM((4, CHUNK), _i32),
        pltpu.VMEM((2, CHUNK, F), _f32),
        pltpu.VMEM((RB + 8, F), _f32),
    ] + [
        pltpu.VMEM((NW * 128 + 16,), _i32),
        pltpu.VMEM((NCHMAX,), _i32),
        pltpu.VMEM((NCHMAX,), _i32),
        pltpu.SemaphoreType.DMA((4,)),
        pltpu.SemaphoreType.DMA((4,)),
        pltpu.SemaphoreType.DMA((2,)),
    ],
)
def _sc_scatter(g_hbm, bsrc_hbm, bdst_hbm, cnt_hbm, zrows_hbm,
                s_out, sidxr, didxr, rows2, acc,
                cntv, cb, cr, semI, semJ, semG):
    c = lax.axis_index("c")
    s = lax.axis_index("s")
    w = c * NS + s
    pltpu.sync_copy(cnt_hbm, cntv.at[pl.ds(0, NW * 128)])
    pltpu.sync_copy(zrows_hbm, acc.at[pl.ds(0, CHUNK)])
    pltpu.sync_copy(zrows_hbm, acc.at[pl.ds(CHUNK, CHUNK)])
    pltpu.sync_copy(zrows_hbm.at[pl.ds(0, RB + 8 - 2 * CHUNK)],
                    acc.at[pl.ds(2 * CHUNK, RB + 8 - 2 * CHUNK)])

    nodebase = w * RB
    lane = lax.iota(_i32, 16)

    @pl.loop(0, NW, init_carry=jnp.int32(0))
    def build(w2, cursor):
        @pl.loop(0, w, init_carry=jnp.int32(0))
        def segoff(b2, o):
            tt = cntv[pl.ds(w2 * 128 + b2, 16)][0]
            return o + (((tt + 7) >> 3) << 3)

        t = cntv[pl.ds(w2 * 128 + w, 16)][0]
        base = w2 * SLAB + segoff
        nch = (t + CHUNK - 1) >> 7

        @pl.loop(0, (nch + 15) >> 4)
        def _(gi):
            kv = gi * 16 + lane
            cb[pl.ds(cursor + gi * 16, 16)] = base + kv * CHUNK
            cr[pl.ds(cursor + gi * 16, 16)] = jnp.minimum(
                jnp.maximum(t - kv * CHUNK, 0), jnp.int32(CHUNK))

        return cursor + nch

    ncht = build

    def issue_idx(j, k):
        b = pl.multiple_of(cb[pl.ds(j, 16)][0], 8)
        pltpu.async_copy(bsrc_hbm.at[pl.ds(b, CHUNK)], sidxr.at[k], semI.at[k])
        pltpu.async_copy(bdst_hbm.at[pl.ds(b, CHUNK)], didxr.at[k], semJ.at[k])

    def wait_idx_issue_gather(k, r):
        pltpu.make_async_copy(bsrc_hbm.at[pl.ds(0, CHUNK)], sidxr.at[k],
                              semI.at[k]).wait()
        pltpu.make_async_copy(bdst_hbm.at[pl.ds(0, CHUNK)], didxr.at[k],
                              semJ.at[k]).wait()
        for q in range(CHUNK // 16):
            v = sidxr[k, pl.ds(q * 16, 16)]
            sidxr[k, pl.ds(q * 16, 16)] = jnp.minimum(
                jnp.maximum(v, 0), jnp.int32(NPAD - 1))
        pltpu.async_copy(g_hbm.at[sidxr.at[k]], rows2.at[r], semG.at[r])

    def process(j, k, r):
        pltpu.make_async_copy(g_hbm.at[sidxr.at[k]], rows2.at[r],
                              semG.at[r]).wait()
        rem = cr[pl.ds(j, 16)][0]

        @pl.loop(0, CHUNK // 16)
        def _(q):
            dv = didxr[k, pl.ds(q * 16, 16)] - nodebase
            dv = jnp.minimum(jnp.maximum(dv, 0), jnp.int32(RB))
            dv = jnp.where(lane + q * 16 < rem, dv, jnp.int32(RB))
            for jj in range(16):
                l = dv[jj]
                e = q * 16 + jj
                for f in range(F // 16):
                    acc[l, pl.ds(f * 16, 16)] = (
                        acc[l, pl.ds(f * 16, 16)]
                        + rows2[r, e, pl.ds(f * 16, 16)])

    for k in range(4):
        @pl.when(k < ncht)
        def _(k=k):
            issue_idx(k, k)
    for r in range(2):
        @pl.when(r < ncht)
        def _(r=r):
            wait_idx_issue_gather(r, r)

    @pl.loop(0, (ncht + 3) >> 2)
    def _(o):
        for ph in range(4):
            j = o * 4 + ph

            @pl.when(j < ncht)
            def _(j=j, ph=ph):
                process(j, ph, ph % 2)

                @pl.when(j + 4 < ncht)
                def _():
                    issue_idx(j + 4, ph)

                @pl.when(j + 2 < ncht)
                def _():
                    wait_idx_issue_gather((ph + 2) % 4, ph % 2)

    pltpu.sync_copy(acc.at[pl.ds(0, RB)], s_out.at[pl.ds(nodebase, RB)])


@functools.partial(
    pl.kernel,
    out_type=jax.ShapeDtypeStruct((1024, F), _f32),
    mesh=_mesh,
    scratch_types=[
        pltpu.VMEM((32, F), _f32),
        pltpu.VMEM((32,), _i32),
        pltpu.SemaphoreType.DMA,
    ],
)
def _sc_gather_batch(feat_hbm, bidx_hbm, out_hbm, rows, bv, sem):
    c = lax.axis_index("c")
    s = lax.axis_index("s")
    pltpu.sync_copy(bidx_hbm.at[c, s], bv)
    pltpu.async_copy(feat_hbm.at[bv], rows, sem).wait()
    pltpu.sync_copy(rows, out_hbm.at[pl.ds(c * 512 + s * 32, 32)])


def _tc_first(deg, x, W1):
    def body(deg_ref, x_ref, w_ref, dis_ref, g_ref):
        i = pl.program_id(0)
        row = i * 128 + lax.broadcasted_iota(_i32, (128, 1), 0)
        dis = jnp.where(row < N, lax.rsqrt(deg_ref[:, 0:1] + 1.0), 0.0)
        dis_ref[...] = dis
        g_ref[...] = jnp.dot(x_ref[...], w_ref[...],
                             preferred_element_type=_f32) * dis

    return pl.pallas_call(
        body,
        grid=(NPAD // 128,),
        in_specs=[
            pl.BlockSpec((128, 16), lambda i: (i, 0)),
            pl.BlockSpec((128, F), lambda i: (i, 0)),
            pl.BlockSpec((F, F), lambda i: (0, 0)),
        ],
        out_specs=[
            pl.BlockSpec((128, 1), lambda i: (i, 0)),
            pl.BlockSpec((128, F), lambda i: (i, 0)),
        ],
        out_shape=[
            jax.ShapeDtypeStruct((NPAD, 1), _f32),
            jax.ShapeDtypeStruct((NPAD, F), _f32),
        ],
    )(deg, x, W1)


def _tc_mid(sacc, g, dis, b, W):
    def body(s_ref, g_ref, dis_ref, b_ref, w_ref, out_ref):
        h = jax.nn.relu(dis_ref[...] * (s_ref[...] + g_ref[...]) + b_ref[...])
        out_ref[...] = jnp.dot(h, w_ref[...],
                               preferred_element_type=_f32) * dis_ref[...]

    return pl.pallas_call(
        body,
        grid=(NPAD // 128,),
        in_specs=[
            pl.BlockSpec((128, F), lambda i: (i, 0)),
            pl.BlockSpec((128, F), lambda i: (i, 0)),
            pl.BlockSpec((128, 1), lambda i: (i, 0)),
            pl.BlockSpec((1, F), lambda i: (0, 0)),
            pl.BlockSpec((F, F), lambda i: (0, 0)),
        ],
        out_specs=pl.BlockSpec((128, F), lambda i: (i, 0)),
        out_shape=jax.ShapeDtypeStruct((NPAD, F), _f32),
    )(sacc, g, dis, b, W)


def _tc_last(sacc, g, dis, b):
    def body(s_ref, g_ref, dis_ref, b_ref, out_ref):
        out_ref[...] = jax.nn.relu(
            dis_ref[...] * (s_ref[...] + g_ref[...]) + b_ref[...])

    return pl.pallas_call(
        body,
        grid=(NPAD // 128,),
        in_specs=[
            pl.BlockSpec((128, F), lambda i: (i, 0)),
            pl.BlockSpec((128, F), lambda i: (i, 0)),
            pl.BlockSpec((128, 1), lambda i: (i, 0)),
            pl.BlockSpec((1, F), lambda i: (0, 0)),
        ],
        out_specs=pl.BlockSpec((128, F), lambda i: (i, 0)),
        out_shape=jax.ShapeDtypeStruct((NPAD, F), _f32),
    )(sacc, g, dis, b)


def _tc_head(feats, Wlin, blin):
    def body(f_ref, w_ref, b_ref, out_ref, logp_ref):
        t = jax.nn.relu(jnp.dot(f_ref[...], w_ref[...],
                                preferred_element_type=_f32) + b_ref[...])
        col = lax.broadcasted_iota(_i32, (1, 128), 1)
        valid = col < 10
        mx = jnp.max(jnp.where(valid, t, -1e30), axis=1, keepdims=True)
        ex = jnp.where(valid, jnp.exp(t - mx), 0.0)
        lse = jnp.log(jnp.sum(ex, axis=1, keepdims=True))
        out_ref[...] = t
        logp_ref[...] = t - mx - lse

    return pl.pallas_call(
        body,
        grid=(1024 // 128,),
        in_specs=[
            pl.BlockSpec((128, F), lambda i: (i, 0)),
            pl.BlockSpec((F, 128), lambda i: (0, 0)),
            pl.BlockSpec((1, 128), lambda i: (0, 0)),
        ],
        out_specs=[
            pl.BlockSpec((128, 128), lambda i: (i, 0)),
            pl.BlockSpec((128, 128), lambda i: (i, 0)),
        ],
        out_shape=[
            jax.ShapeDtypeStruct((1024, 128), _f32),
            jax.ShapeDtypeStruct((1024, 128), _f32),
        ],
    )(feats, Wlin, blin)


def kernel(x, edge_index, batch_index, W1, b1, W2, b2, Wlin, blin):
    src = edge_index[0].astype(_i32)
    dst = edge_index[1].astype(_i32)
    pad = jnp.full((EPAD - E,), N, dtype=_i32)
    srcf = jnp.concatenate([src, pad]).reshape(NW, EPW)
    dstf = jnp.concatenate([dst, pad]).reshape(NW, EPW)
    xp = jnp.concatenate([x, jnp.zeros((NPAD - N, F), dtype=_f32)])
    bidx = batch_index.astype(_i32).reshape(NC, NS, 32)
    zrows = jnp.zeros((CHUNK, F), dtype=_f32)
    Wlp = jnp.concatenate(
        [Wlin, jnp.zeros((F, 128 - Wlin.shape[1]), dtype=_f32)], axis=1)
    blp = jnp.concatenate(
        [blin, jnp.zeros((128 - blin.shape[0],), dtype=_f32)]).reshape(1, 128)
    b1r = b1.reshape(1, F)
    b2r = b2.reshape(1, F)

    bsrc, bdst, cnt = _sc_binsort(srcf, dstf)
    cntf = cnt.reshape(NW * 128)
    deg = _sc_degree(bdst, cntf)
    dis, g1 = _tc_first(deg, xp, W1)
    s1 = _sc_scatter(g1, bsrc, bdst, cntf, zrows)
    g2 = _tc_mid(s1, g1, dis, b1r, W2)
    s2 = _sc_scatter(g2, bsrc, bdst, cntf, zrows)
    feats = _tc_last(s2, g2, dis, b2r)
    feats_sel = _sc_gather_batch(feats, bidx)
    outp, logpp = _tc_head(feats_sel, Wlp, blp)
    return (logpp[:, :10], outp[:, :10], feats_sel)
